# Initial kernel scaffold; baseline (speedup 1.0000x reference)
import jax
import jax.numpy as jnp
from jax.experimental import pallas as pl


def kernel(position_ids, table):
    raise NotImplementedError("write your pallas kernel here")


# speedup vs baseline: 2.1887x; 2.1887x over previous
import functools

import jax
import jax.numpy as jnp
from jax import lax
from jax.experimental import pallas as pl
from jax.experimental.pallas import tpu as pltpu
from jax.experimental.pallas import tpu_sc as plsc

_info = plsc.get_sparse_core_info()
_NC, _NS = _info.num_cores, _info.num_subcores
_NW = _NC * _NS


def _make_gather(V, D, B):
    assert B % (8 * _NW) == 0
    b_per_w = B // _NW
    C = 64
    assert b_per_w % C == 0
    n_chunks = b_per_w // C
    mesh = plsc.VectorSubcoreMesh(core_axis_name="c", subcore_axis_name="s")

    @functools.partial(
        pl.kernel,
        mesh=mesh,
        out_type=jax.ShapeDtypeStruct((B, D), jnp.float32),
        scratch_types=[
            pltpu.VMEM((b_per_w,), jnp.int32),
            pltpu.VMEM((C, D), jnp.float32),
            pltpu.SemaphoreType.DMA,
        ],
    )
    def gather_kernel(table_hbm, idx_hbm, out_hbm, idx_v, buf, sem):
        wid = lax.axis_index("s") * _NC + lax.axis_index("c")
        base = wid * b_per_w
        pltpu.sync_copy(idx_hbm.at[pl.ds(base, b_per_w)], idx_v)

        def chunk_body(g, carry):
            off = g * C
            pltpu.async_copy(
                table_hbm.at[idx_v.at[pl.ds(off, C)]], buf, sem
            ).wait()
            pltpu.sync_copy(buf, out_hbm.at[pl.ds(base + off, C)])
            return carry

        lax.fori_loop(0, n_chunks, chunk_body, 0)

    return gather_kernel


def kernel(position_ids, table):
    Bt, S = position_ids.shape
    V, D = table.shape
    idx = position_ids.reshape(Bt * S).astype(jnp.int32)
    out = _make_gather(V, D, Bt * S)(table, idx)
    return out.reshape(Bt, S, D)


# speedup vs baseline: 2.3869x; 1.0905x over previous
import functools

import jax
import jax.numpy as jnp
from jax import lax
from jax.experimental import pallas as pl
from jax.experimental.pallas import tpu as pltpu
from jax.experimental.pallas import tpu_sc as plsc

_info = plsc.get_sparse_core_info()
_NC, _NS = _info.num_cores, _info.num_subcores
_NW = _NC * _NS


def _make_gather(V, D, B, C=32, NBUF=2):
    assert B % (8 * _NW) == 0
    b_per_w = B // _NW
    assert b_per_w % (C * NBUF) == 0
    n_chunks = b_per_w // C
    n_groups = n_chunks // NBUF
    mesh = plsc.VectorSubcoreMesh(core_axis_name="c", subcore_axis_name="s")

    @functools.partial(
        pl.kernel,
        mesh=mesh,
        out_type=jax.ShapeDtypeStruct((B, D), jnp.float32),
        scratch_types=[
            pltpu.VMEM((b_per_w,), jnp.int32),
        ]
        + [pltpu.VMEM((C, D), jnp.float32) for _ in range(NBUF)]
        + [pltpu.SemaphoreType.DMA((NBUF,)), pltpu.SemaphoreType.DMA((NBUF,))],
    )
    def gather_kernel(table_hbm, idx_hbm, out_hbm, idx_v, *rest):
        bufs, (gsem, osem) = rest[:NBUF], rest[NBUF:]
        wid = lax.axis_index("s") * _NC + lax.axis_index("c")
        base = wid * b_per_w
        pltpu.sync_copy(idx_hbm.at[pl.ds(base, b_per_w)], idx_v)

        def gather_dma(chunk, j):
            return pltpu.make_async_copy(
                table_hbm.at[idx_v.at[pl.ds(chunk * C, C)]], bufs[j], gsem.at[j]
            )

        def out_dma(chunk, j):
            return pltpu.make_async_copy(
                bufs[j], out_hbm.at[pl.ds(base + chunk * C, C)], osem.at[j]
            )

        for j in range(NBUF):
            gather_dma(j, j).start()

        def group_body(g, carry):
            for j in range(NBUF):
                chunk = g * NBUF + j
                gather_dma(chunk, j).wait()
                out_dma(chunk, j).start()
                out_dma(chunk, j).wait()
                gather_dma(chunk + NBUF, j).start()
            return carry

        lax.fori_loop(0, n_groups - 1, group_body, 0)

        for j in range(NBUF):
            chunk = (n_groups - 1) * NBUF + j
            gather_dma(chunk, j).wait()
            out_dma(chunk, j).start()
        for j in range(NBUF):
            chunk = (n_groups - 1) * NBUF + j
            out_dma(chunk, j).wait()

    return gather_kernel


def kernel(position_ids, table):
    Bt, S = position_ids.shape
    V, D = table.shape
    idx = position_ids.reshape(Bt * S).astype(jnp.int32)
    out = _make_gather(V, D, Bt * S)(table, idx)
    return out.reshape(Bt, S, D)


# speedup vs baseline: 2.3960x; 1.0038x over previous
import functools

import jax
import jax.numpy as jnp
from jax import lax
from jax.experimental import pallas as pl
from jax.experimental.pallas import tpu as pltpu
from jax.experimental.pallas import tpu_sc as plsc

_info = plsc.get_sparse_core_info()
_NC, _NS = _info.num_cores, _info.num_subcores
_NW = _NC * _NS


def _make_gather(V, D, B, C=32, NBUF=2):
    assert B % (8 * _NW) == 0
    b_per_w = B // _NW
    assert b_per_w % (C * NBUF) == 0
    n_chunks = b_per_w // C
    n_groups = n_chunks // NBUF
    mesh = plsc.VectorSubcoreMesh(core_axis_name="c", subcore_axis_name="s")

    @functools.partial(
        pl.kernel,
        mesh=mesh,
        out_type=jax.ShapeDtypeStruct((B, D), jnp.float32),
        scratch_types=[
            pltpu.VMEM((b_per_w,), jnp.int32),
        ]
        + [pltpu.VMEM((C, D), jnp.float32) for _ in range(NBUF)]
        + [pltpu.SemaphoreType.DMA((NBUF,)), pltpu.SemaphoreType.DMA((NBUF,))],
    )
    def gather_kernel(table_hbm, idx_hbm, out_hbm, idx_v, *rest):
        bufs, (gsem, osem) = rest[:NBUF], rest[NBUF:]
        wid = lax.axis_index("s") * _NC + lax.axis_index("c")
        base = wid * b_per_w
        pltpu.sync_copy(idx_hbm.at[pl.ds(base, b_per_w)], idx_v)

        def gather_dma(chunk, j):
            return pltpu.make_async_copy(
                table_hbm.at[idx_v.at[pl.ds(chunk * C, C)]], bufs[j], gsem.at[j]
            )

        def out_dma(chunk, j):
            return pltpu.make_async_copy(
                bufs[j], out_hbm.at[pl.ds(base + chunk * C, C)], osem.at[j]
            )

        for j in range(NBUF):
            gather_dma(j, j).start()

        def group_body(g, carry):
            for j in range(NBUF):
                chunk = g * NBUF + j
                gather_dma(chunk, j).wait()
                out_dma(chunk, j).start()
                out_dma(chunk, j).wait()
                gather_dma(chunk + NBUF, j).start()
            return carry

        lax.fori_loop(0, n_groups - 1, group_body, 0)

        for j in range(NBUF):
            chunk = (n_groups - 1) * NBUF + j
            gather_dma(chunk, j).wait()
            out_dma(chunk, j).start()
        for j in range(NBUF):
            chunk = (n_groups - 1) * NBUF + j
            out_dma(chunk, j).wait()

    return gather_kernel


def kernel(position_ids, table):
    Bt, S = position_ids.shape
    V, D = table.shape
    idx = position_ids.reshape(Bt * S).astype(jnp.int32)
    out = _make_gather(V, D, Bt * S, C=16, NBUF=4)(table, idx)
    return out.reshape(Bt, S, D)
